# baseline (speedup 1.0000x reference)
import functools

import jax
import jax.numpy as jnp
from jax import lax
from jax.experimental import pallas as pl
from jax.experimental.pallas import tpu as pltpu
from jax.experimental.pallas import tpu_sc as plsc

NUM_EMB = 1_000_000
DIM = 64
B_TOTAL = 16384 * 20
LANE = 128
N_WORKERS = 32
ROWS_TOTAL = B_TOTAL // LANE
ROWS_PER_W = ROWS_TOTAL // N_WORKERS
KCH = 4
N_STEPS = ROWS_PER_W // KCH


def _sc_gather(x2, table):
    mesh = plsc.VectorSubcoreMesh(core_axis_name="c", subcore_axis_name="s")

    @functools.partial(
        pl.kernel,
        out_type=jax.ShapeDtypeStruct((ROWS_TOTAL, LANE, DIM), jnp.float32),
        mesh=mesh,
        scratch_types=[
            pltpu.VMEM((KCH, LANE), jnp.int32),
            pltpu.VMEM((KCH, LANE, DIM), jnp.float32),
            pltpu.SemaphoreType.DMA,
        ],
        compiler_params=pltpu.CompilerParams(use_tc_tiling_on_sc=False),
    )
    def k(x_hbm, tab_hbm, out_hbm, idx_v, rows_v, gsem):
        wid = lax.axis_index("s") * 2 + lax.axis_index("c")
        row0 = wid * ROWS_PER_W

        def step(g, carry):
            r = row0 + g * KCH
            pltpu.sync_copy(x_hbm.at[pl.ds(r, KCH)], idx_v)
            for j in range(KCH):
                pltpu.async_copy(tab_hbm.at[idx_v.at[j]], rows_v.at[j], gsem)
            for j in range(KCH):
                pltpu.make_async_copy(
                    tab_hbm.at[idx_v.at[j]], rows_v.at[j], gsem
                ).wait()
            pltpu.sync_copy(rows_v, out_hbm.at[pl.ds(r, KCH)])
            return carry

        lax.fori_loop(0, N_STEPS, step, 0)

    return k(x2, table)


def kernel(x, A):
    x2 = x.reshape(ROWS_TOTAL, LANE).astype(jnp.int32)
    out = _sc_gather(x2, A)
    return out.reshape(16384, 20, DIM)


# speedup vs baseline: 1.0154x; 1.0154x over previous
import functools

import jax
import jax.numpy as jnp
from jax import lax
from jax.experimental import pallas as pl
from jax.experimental.pallas import tpu as pltpu
from jax.experimental.pallas import tpu_sc as plsc

NUM_EMB = 1_000_000
DIM = 64
B_TOTAL = 16384 * 20
LANE = 128
N_WORKERS = 32
ROWS_TOTAL = B_TOTAL // LANE
ROWS_PER_W = ROWS_TOTAL // N_WORKERS
KCH = 4
N_CHUNKS = ROWS_PER_W // KCH


def _sc_gather(x2, table):
    mesh = plsc.VectorSubcoreMesh(core_axis_name="c", subcore_axis_name="s")

    @functools.partial(
        pl.kernel,
        out_type=jax.ShapeDtypeStruct((ROWS_TOTAL, LANE, DIM), jnp.float32),
        mesh=mesh,
        scratch_types=[
            pltpu.VMEM((KCH, LANE), jnp.int32),
            pltpu.VMEM((KCH, LANE), jnp.int32),
            pltpu.VMEM((KCH, LANE, DIM), jnp.float32),
            pltpu.VMEM((KCH, LANE, DIM), jnp.float32),
            pltpu.SemaphoreType.DMA,
            pltpu.SemaphoreType.DMA,
        ],
        compiler_params=pltpu.CompilerParams(use_tc_tiling_on_sc=False),
    )
    def k(x_hbm, tab_hbm, out_hbm, idx0, idx1, rows0, rows1, sem0, sem1):
        wid = lax.axis_index("s") * 2 + lax.axis_index("c")
        row0 = wid * ROWS_PER_W
        idx_b = (idx0, idx1)
        rows_b = (rows0, rows1)
        sem_b = (sem0, sem1)

        def fire(g, b):
            r = row0 + g * KCH
            pltpu.sync_copy(x_hbm.at[pl.ds(r, KCH)], idx_b[b])
            for j in range(KCH):
                pltpu.async_copy(
                    tab_hbm.at[idx_b[b].at[j]], rows_b[b].at[j], sem_b[b]
                )

        def drain_store(g, b):
            for j in range(KCH):
                pltpu.make_async_copy(
                    tab_hbm.at[idx_b[b].at[j]], rows_b[b].at[j], sem_b[b]
                ).wait()
            pltpu.sync_copy(rows_b[b], out_hbm.at[pl.ds(row0 + g * KCH, KCH)])

        fire(0, 0)

        def outer(gg, carry):
            for b in range(2):
                g = gg * 2 + b

                @pl.when(g + 1 < N_CHUNKS)
                def _():
                    fire(g + 1, 1 - b)

                drain_store(g, b)
            return carry

        lax.fori_loop(0, N_CHUNKS // 2, outer, 0)

    return k(x2, table)


def kernel(x, A):
    x2 = x.reshape(ROWS_TOTAL, LANE).astype(jnp.int32)
    out = _sc_gather(x2, A)
    return out.reshape(16384, 20, DIM)


# speedup vs baseline: 1.0581x; 1.0421x over previous
import functools

import jax
import jax.numpy as jnp
from jax import lax
from jax.experimental import pallas as pl
from jax.experimental.pallas import tpu as pltpu
from jax.experimental.pallas import tpu_sc as plsc

NUM_EMB = 1_000_000
DIM = 64
B_TOTAL = 16384 * 20
LANE = 128
N_WORKERS = 32
ROWS_TOTAL = B_TOTAL // LANE
ROWS_PER_W = ROWS_TOTAL // N_WORKERS
KCH = 4
N_CHUNKS = ROWS_PER_W // KCH


def _sc_gather(x2, table):
    mesh = plsc.VectorSubcoreMesh(core_axis_name="c", subcore_axis_name="s")

    @functools.partial(
        pl.kernel,
        out_type=jax.ShapeDtypeStruct((ROWS_TOTAL, LANE, DIM), jnp.float32),
        mesh=mesh,
        scratch_types=[
            pltpu.VMEM((KCH, LANE), jnp.int32),
            pltpu.VMEM((KCH, LANE), jnp.int32),
            pltpu.VMEM((KCH, LANE, DIM), jnp.float32),
            pltpu.VMEM((KCH, LANE, DIM), jnp.float32),
            pltpu.SemaphoreType.DMA,
            pltpu.SemaphoreType.DMA,
        ],
        compiler_params=pltpu.CompilerParams(use_tc_tiling_on_sc=False),
    )
    def k(x_hbm, tab_hbm, out_hbm, idx0, idx1, rows0, rows1, sem0, sem1):
        wid = lax.axis_index("s") * 2 + lax.axis_index("c")
        row0 = wid * ROWS_PER_W
        idx_b = (idx0, idx1)
        rows_b = (rows0, rows1)
        sem_b = (sem0, sem1)

        def fire(g, b):
            r = row0 + g * KCH
            pltpu.sync_copy(x_hbm.at[pl.ds(r, KCH)], idx_b[b])
            for j in range(KCH):
                pltpu.async_copy(
                    tab_hbm.at[idx_b[b].at[j]], rows_b[b].at[j], sem_b[b]
                )

        def drain_store(g, b):
            for j in range(KCH):
                pltpu.make_async_copy(
                    tab_hbm.at[idx_b[b].at[j]], rows_b[b].at[j], sem_b[b]
                ).wait()
            pltpu.sync_copy(rows_b[b], out_hbm.at[pl.ds(row0 + g * KCH, KCH)])

        fire(0, 0)

        def outer(gg, carry):
            for b in range(2):
                g = gg * 2 + b

                @pl.when(g + 1 < N_CHUNKS)
                def _():
                    fire(g + 1, 1 - b)

                drain_store(g, b)
            return carry

        lax.fori_loop(0, N_CHUNKS // 2, outer, 0)

    return k(x2, table)


def kernel(x, A):
    x2 = x.T.reshape(ROWS_TOTAL, LANE).astype(jnp.int32)
    out = _sc_gather(x2, A)
    return out.reshape(20, 16384, DIM).transpose(1, 0, 2)
